# baseline (speedup 1.0000x reference)
import functools

import jax
import jax.numpy as jnp
from jax import lax
from jax.experimental import pallas as pl
from jax.experimental.pallas import tpu as pltpu
from jax.experimental.pallas import tpu_sc as plsc

_XL, _YL, _ZL = 511, 511, 255
_WATER = 0.08

_N = 1048576
_NC = 2
_NS = 16
_NW = _NC * _NS
_P = _N // _NW
_C = 8192
_NCH = _P // _C
_G = 512
_R = _C // _G
_U = _G // 16


def _sc_body(xs, ys, zs, par, img, out, xv, yv, zv, pv, idxb, vb, gb, sem):
    wid = lax.axis_index("s") * _NC + lax.axis_index("c")
    base = wid * _P
    pltpu.sync_copy(par, pv)

    def chunk(k, _):
        off = base + k * _C
        pltpu.sync_copy(xs.at[pl.ds(off, _C)], xv)
        pltpu.sync_copy(ys.at[pl.ds(off, _C)], yv)
        pltpu.sync_copy(zs.at[pl.ds(off, _C)], zv)

        def ixloop(j, _):
            for u in range(_U):
                s = pl.ds(j * _G + u * 16, 16)
                x = xv[s]
                y = yv[s]
                z = zv[s]
                ixi = ((x + pv[0]) * pv[3]).astype(jnp.int32)
                iyi = ((y + pv[1]) * pv[4]).astype(jnp.int32)
                izi = ((z + pv[2]) * pv[5]).astype(jnp.int32)
                m = ((ixi < 0) | (iyi < 0) | (izi < 0)
                     | (ixi > _XL) | (iyi > _YL) | (izi > _ZL))
                lin = (ixi * 131072 + (iyi >> 3) * 2048 + (izi >> 7) * 1024
                       + (iyi & 7) * 128 + (izi & 127))
                idxb[s] = jnp.where(m, 0, lin)
                vb[s] = jnp.where(m, jnp.float32(0.0), jnp.float32(1.0))
            blk = pl.ds(j * _G, _G)
            pltpu.async_copy(img.at[idxb.at[blk]], gb.at[blk], sem)
            return 0

        lax.fori_loop(0, _R, ixloop, 0)

        def drain(r, _):
            blk = pl.ds(r * _G, _G)
            pltpu.make_async_copy(img.at[idxb.at[blk]], gb.at[blk], sem).wait()
            return 0

        lax.fori_loop(0, _R, drain, 0)

        def trloop(j, _):
            for u in range(_U):
                s = pl.ds(j * _G + u * 16, 16)
                t = jnp.maximum(gb[s], jnp.float32(-1000.0)) * jnp.float32(1e-3)
                gb[s] = (t + jnp.float32(1.0)) * jnp.float32(_WATER) * vb[s]
            return 0

        lax.fori_loop(0, _R, trloop, 0)
        pltpu.sync_copy(gb, out.at[pl.ds(off, _C)])
        return 0

    lax.fori_loop(0, _NCH, chunk, 0)


_sc_gather = functools.partial(
    pl.kernel,
    out_type=jax.ShapeDtypeStruct((_N,), jnp.float32),
    mesh=plsc.VectorSubcoreMesh(core_axis_name="c", subcore_axis_name="s"),
    scratch_types=[
        pltpu.VMEM((_C,), jnp.float32),
        pltpu.VMEM((_C,), jnp.float32),
        pltpu.VMEM((_C,), jnp.float32),
        pltpu.VMEM((6, 16), jnp.float32),
        pltpu.VMEM((_C,), jnp.int32),
        pltpu.VMEM((_C,), jnp.float32),
        pltpu.VMEM((_C,), jnp.float32),
        pltpu.SemaphoreType.DMA,
    ],
)(_sc_body)


def kernel(xyz, img, ct_size):
    pts = xyz[0]
    xs = pts[:, 0]
    ys = pts[:, 1]
    zs = pts[:, 2]
    img_flat = (img.reshape(512, 64, 8, 2, 128)
                .transpose(0, 1, 3, 2, 4).reshape(-1))
    half = ct_size / 2.0
    lims = jnp.array([_XL, _YL, _ZL], dtype=jnp.float32)
    scale = lims / ct_size
    par = jnp.broadcast_to(
        jnp.concatenate([half, scale]).astype(jnp.float32).reshape(6, 1),
        (6, 16),
    )
    sigma = _sc_gather(xs, ys, zs, par, img_flat)
    rgb = jnp.ones((1, _N, 3), jnp.float32)
    return jnp.concatenate((rgb, sigma.reshape(1, _N, 1)), axis=-1)


# speedup vs baseline: 1.0005x; 1.0005x over previous
import functools

import jax
import jax.numpy as jnp
from jax import lax
from jax.experimental import pallas as pl
from jax.experimental.pallas import tpu as pltpu
from jax.experimental.pallas import tpu_sc as plsc

_XL, _YL, _ZL = 511, 511, 255
_WATER = 0.08

_N = 1048576
_NC = 2
_NS = 16
_NW = _NC * _NS
_P = _N // _NW
_C = 8192
_NCH = _P // _C
_G = 128
_R = _C // _G
_U = _G // 16


def _sc_body(xs, ys, zs, par, img, out,
             xv, yv, zv, pv, i0, v0, g0, i1, v1, g1, s0, s1):
    wid = lax.axis_index("s") * _NC + lax.axis_index("c")
    base = wid * _P
    pltpu.sync_copy(par, pv)
    banks = ((i0, v0, g0, s0), (i1, v1, g1, s1))

    def compute_and_fire(k, idxb, vb, gb, sem):
        off = base + k * _C
        pltpu.sync_copy(xs.at[pl.ds(off, _C)], xv)
        pltpu.sync_copy(ys.at[pl.ds(off, _C)], yv)
        pltpu.sync_copy(zs.at[pl.ds(off, _C)], zv)

        def ixloop(j, _):
            for u in range(_U):
                s = pl.ds(j * _G + u * 16, 16)
                x = xv[s]
                y = yv[s]
                z = zv[s]
                ixi = ((x + pv[0]) * pv[3]).astype(jnp.int32)
                iyi = ((y + pv[1]) * pv[4]).astype(jnp.int32)
                izi = ((z + pv[2]) * pv[5]).astype(jnp.int32)
                m = ((ixi < 0) | (iyi < 0) | (izi < 0)
                     | (ixi > _XL) | (iyi > _YL) | (izi > _ZL))
                lin = (ixi * 131072 + (iyi >> 3) * 2048 + (izi >> 7) * 1024
                       + (iyi & 7) * 128 + (izi & 127))
                idxb[s] = jnp.where(m, 0, lin)
                vb[s] = jnp.where(m, jnp.float32(0.0), jnp.float32(1.0))
            blk = pl.ds(j * _G, _G)
            pltpu.async_copy(img.at[idxb.at[blk]], gb.at[blk], sem)
            return 0

        lax.fori_loop(0, _R, ixloop, 0)

    def drain_and_emit(k, idxb, vb, gb, sem):
        off = base + k * _C

        def drain(r, _):
            blk = pl.ds(r * _G, _G)
            pltpu.make_async_copy(img.at[idxb.at[blk]], gb.at[blk], sem).wait()
            return 0

        lax.fori_loop(0, _R, drain, 0)

        def trloop(j, _):
            for u in range(_U):
                s = pl.ds(j * _G + u * 16, 16)
                t = jnp.maximum(gb[s], jnp.float32(-1000.0)) * jnp.float32(1e-3)
                gb[s] = (t + jnp.float32(1.0)) * jnp.float32(_WATER) * vb[s]
            return 0

        lax.fori_loop(0, _R, trloop, 0)
        pltpu.sync_copy(gb, out.at[pl.ds(off, _C)])

    for k in range(_NCH):
        compute_and_fire(k, *banks[k % 2])
        if k > 0:
            drain_and_emit(k - 1, *banks[(k - 1) % 2])
    drain_and_emit(_NCH - 1, *banks[(_NCH - 1) % 2])


_sc_gather = functools.partial(
    pl.kernel,
    out_type=jax.ShapeDtypeStruct((_N,), jnp.float32),
    mesh=plsc.VectorSubcoreMesh(core_axis_name="c", subcore_axis_name="s"),
    scratch_types=[
        pltpu.VMEM((_C,), jnp.float32),
        pltpu.VMEM((_C,), jnp.float32),
        pltpu.VMEM((_C,), jnp.float32),
        pltpu.VMEM((6, 16), jnp.float32),
        pltpu.VMEM((_C,), jnp.int32),
        pltpu.VMEM((_C,), jnp.float32),
        pltpu.VMEM((_C,), jnp.float32),
        pltpu.VMEM((_C,), jnp.int32),
        pltpu.VMEM((_C,), jnp.float32),
        pltpu.VMEM((_C,), jnp.float32),
        pltpu.SemaphoreType.DMA,
        pltpu.SemaphoreType.DMA,
    ],
)(_sc_body)


def kernel(xyz, img, ct_size):
    pts = xyz[0]
    xs = pts[:, 0]
    ys = pts[:, 1]
    zs = pts[:, 2]
    img_flat = (img.reshape(512, 64, 8, 2, 128)
                .transpose(0, 1, 3, 2, 4).reshape(-1))
    half = ct_size / 2.0
    lims = jnp.array([_XL, _YL, _ZL], dtype=jnp.float32)
    scale = lims / ct_size
    par = jnp.broadcast_to(
        jnp.concatenate([half, scale]).astype(jnp.float32).reshape(6, 1),
        (6, 16),
    )
    sigma = _sc_gather(xs, ys, zs, par, img_flat)
    rgb = jnp.ones((1, _N, 3), jnp.float32)
    return jnp.concatenate((rgb, sigma.reshape(1, _N, 1)), axis=-1)
